# baseline (speedup 1.0000x reference)
import functools

import jax
import jax.numpy as jnp
from jax import lax
from jax.experimental import pallas as pl
from jax.experimental.pallas import tpu as pltpu
from jax.experimental.pallas import tpu_sc as plsc


def kernel(x, emb):
    seq_len = x.shape[1]
    d_model = emb.shape[1]
    info = plsc.get_sparse_core_info()
    nc, ns = info.num_cores, info.num_subcores
    nw = nc * ns
    rows_per_w = seq_len // nw
    mesh = plsc.VectorSubcoreMesh(core_axis_name="c", subcore_axis_name="s")

    @functools.partial(
        pl.kernel,
        mesh=mesh,
        out_type=jax.ShapeDtypeStruct((seq_len, d_model), jnp.float32),
        scratch_types=[pltpu.SemaphoreType.DMA],
    )
    def copy_k(emb_hbm, out_hbm, sem):
        wid = lax.axis_index("s") * nc + lax.axis_index("c")
        base = wid * rows_per_w
        pltpu.async_copy(
            emb_hbm.at[pl.ds(base, rows_per_w)],
            out_hbm.at[pl.ds(base, rows_per_w)],
            sem,
        ).wait()

    out = copy_k(emb[:seq_len])
    return out[None]


# speedup vs baseline: 31.3241x; 31.3241x over previous
import functools

import jax
import jax.numpy as jnp
from jax import lax
from jax.experimental import pallas as pl
from jax.experimental.pallas import tpu as pltpu
from jax.experimental.pallas import tpu_sc as plsc


def kernel(x, emb):
    seq_len = x.shape[1]
    d_model = emb.shape[1]
    info = plsc.get_sparse_core_info()
    nc, ns = info.num_cores, info.num_subcores
    nw = nc * ns
    rows_per_w = seq_len // nw
    mesh = plsc.VectorSubcoreMesh(core_axis_name="c", subcore_axis_name="s")

    ch = 16
    nch = rows_per_w // ch

    @functools.partial(
        pl.kernel,
        mesh=mesh,
        out_type=jax.ShapeDtypeStruct((seq_len, d_model), jnp.float32),
        scratch_types=[
            pltpu.VMEM((2, ch, d_model), jnp.float32),
            pltpu.SemaphoreType.DMA,
            pltpu.SemaphoreType.DMA,
            pltpu.SemaphoreType.DMA,
            pltpu.SemaphoreType.DMA,
        ],
    )
    def copy_k(emb_hbm, out_hbm, buf, si0, si1, so0, so1):
        wid = lax.axis_index("s") * nc + lax.axis_index("c")
        base = wid * rows_per_w
        sin = (si0, si1)
        sout = (so0, so1)

        def start_in(i):
            return pltpu.async_copy(
                emb_hbm.at[pl.ds(base + i * ch, ch)], buf.at[i % 2], sin[i % 2]
            )

        def start_out(i):
            return pltpu.async_copy(
                buf.at[i % 2], out_hbm.at[pl.ds(base + i * ch, ch)], sout[i % 2]
            )

        in_cp = [None, None]
        out_cp = [None, None]
        in_cp[0] = start_in(0)
        for i in range(nch):
            s = i % 2
            nxt = (i + 1) % 2
            if i + 1 < nch:
                if out_cp[nxt] is not None:
                    out_cp[nxt].wait()
                in_cp[nxt] = start_in(i + 1)
            in_cp[s].wait()
            out_cp[s] = start_out(i)
        for s in range(2):
            if out_cp[s] is not None:
                out_cp[s].wait()

    out = copy_k(emb[:seq_len])
    return out[None]
